# baseline (speedup 1.0000x reference)
import functools
import math

import jax
import jax.numpy as jnp
from jax import lax
from jax.experimental import pallas as pl
from jax.experimental.pallas import tpu as pltpu
from jax.experimental.pallas import tpu_sc as plsc

VOCAB = 1000000
DIM = 64
BATCH = 16384
HIST = 50
SCALE = math.sqrt(DIM)

_info = plsc.get_sparse_core_info()
NC = _info.num_cores
NS = _info.num_subcores
NW = NC * NS
B_PER_W = BATCH // NW
JPAIRS = BATCH // 256
JP_PER_W = JPAIRS // NW
N_UNITS = HIST * JP_PER_W
NBUF = 4

_mesh = plsc.VectorSubcoreMesh(core_axis_name="c", subcore_axis_name="s")


@functools.partial(
    pl.kernel,
    mesh=_mesh,
    out_type=jax.ShapeDtypeStruct((HIST, DIM // 8, JPAIRS, 2048), jnp.float32),
    scratch_types=[
        pltpu.VMEM((HIST, B_PER_W), jnp.int32),
        pltpu.VMEM((NBUF, 256, DIM), jnp.float32),
        pltpu.VMEM((2, DIM // 8, 2048), jnp.float32),
        pltpu.SemaphoreType.DMA,
        pltpu.SemaphoreType.DMA,
        pltpu.SemaphoreType.DMA,
        pltpu.SemaphoreType.DMA,
        pltpu.SemaphoreType.DMA,
        pltpu.SemaphoreType.DMA,
    ],
    compiler_params=pltpu.CompilerParams(
        use_tc_tiling_on_sc=False, needs_layout_passes=False
    ),
)
def _embed_sc(idx_hbm, table_hbm, out_hbm, idx_v, rows_v, patch_v,
              g0, g1, g2, g3, s0, s1):
    wid = lax.axis_index("s") * NC + lax.axis_index("c")
    base_b = wid * B_PER_W
    jp_base = wid * JP_PER_W
    gsems = (g0, g1, g2, g3)
    osems = (s0, s1)

    pltpu.sync_copy(idx_hbm.at[:, pl.ds(base_b, B_PER_W)], idx_v)

    lane = lax.iota(jnp.int32, 16)
    i_base = lane >> 3
    col_base = (lane & 7) * 128

    def unit_hj(u):
        return u // JP_PER_W, u % JP_PER_W

    def start_gather(u, b):
        h, jl = unit_hj(u)
        pltpu.async_copy(
            table_hbm.at[idx_v.at[h, pl.ds(jl * 256, 256)]],
            rows_v.at[b], gsems[b],
        )

    def wait_gather(b):
        pltpu.make_async_copy(
            table_hbm.at[idx_v.at[0, pl.ds(0, 256)]], rows_v.at[b], gsems[b]
        ).wait()

    def scatter_scale(b, pb):
        rbuf = rows_v.at[b]
        pbuf = patch_v.at[pb]

        @plsc.parallel_loop(0, 256, step=1, unroll=8)
        def _(r):
            inner = col_base + ((r >> 7) << 10) + (r & 127)
            for k in range(DIM // 16):
                v = rbuf[r, pl.ds(16 * k, 16)] * SCALE
                plsc.store_scatter(pbuf, [i_base + 2 * k, inner], v)

    def out_slice(u):
        h, jl = unit_hj(u)
        return out_hbm.at[h, :, jp_base + jl]

    for u0 in range(NBUF - 1):
        start_gather(u0, u0)

    def quad_body(t, carry):
        for b in range(NBUF):
            u = t * NBUF + b
            pb = b % 2

            @pl.when(u + NBUF - 1 < N_UNITS)
            def _():
                start_gather(u + NBUF - 1, (b + NBUF - 1) % NBUF)

            wait_gather(b)

            @pl.when(u >= 2)
            def _():
                pltpu.make_async_copy(
                    patch_v.at[pb], out_slice(u), osems[pb]
                ).wait()

            scatter_scale(b, pb)
            pltpu.async_copy(patch_v.at[pb], out_slice(u), osems[pb])
        return carry

    lax.fori_loop(0, N_UNITS // NBUF, quad_body, 0)

    pltpu.make_async_copy(patch_v.at[0], out_slice(0), osems[0]).wait()
    pltpu.make_async_copy(patch_v.at[1], out_slice(1), osems[1]).wait()


def kernel(input, table):
    out4 = _embed_sc(input.T, table)
    o5 = out4.reshape(HIST, DIM // 8, JPAIRS * 2, 8, 128)
    o6 = o5.transpose(2, 4, 0, 1, 3)
    return o6.reshape(BATCH, HIST, DIM)


# speedup vs baseline: 1.7435x; 1.7435x over previous
import functools
import math

import jax
import jax.numpy as jnp
from jax import lax
from jax.experimental import pallas as pl
from jax.experimental.pallas import tpu as pltpu
from jax.experimental.pallas import tpu_sc as plsc

VOCAB = 1000000
DIM = 64
BATCH = 16384
HIST = 50
SCALE = math.sqrt(DIM)

_info = plsc.get_sparse_core_info()
NC = _info.num_cores
NS = _info.num_subcores
NW = NC * NS
B_PER_W = BATCH // NW
JCOLS = BATCH // 128
J_PER_W = JCOLS // NW
N_UNITS = HIST * J_PER_W
NBUF = 4

_mesh = plsc.VectorSubcoreMesh(core_axis_name="c", subcore_axis_name="s")


@functools.partial(
    pl.kernel,
    mesh=_mesh,
    out_type=jax.ShapeDtypeStruct((HIST, DIM // 8, JCOLS, 8, 128),
                                  jnp.float32),
    scratch_types=[
        pltpu.VMEM((HIST, B_PER_W), jnp.int32),
        pltpu.VMEM((NBUF, 128, DIM), jnp.float32),
        pltpu.VMEM((2, DIM // 8, 8, 129), jnp.float32),
        pltpu.SemaphoreType.DMA,
        pltpu.SemaphoreType.DMA,
        pltpu.SemaphoreType.DMA,
        pltpu.SemaphoreType.DMA,
        pltpu.SemaphoreType.DMA,
        pltpu.SemaphoreType.DMA,
    ],
    compiler_params=pltpu.CompilerParams(
        use_tc_tiling_on_sc=False, needs_layout_passes=False
    ),
)
def _embed_sc(idx_hbm, table_hbm, out_hbm, idx_v, rows_v, patch_v,
              g0, g1, g2, g3, s0, s1):
    wid = lax.axis_index("s") * NC + lax.axis_index("c")
    base_b = wid * B_PER_W
    jg_base = wid * J_PER_W
    gsems = (g0, g1, g2, g3)
    osems = (s0, s1)

    pltpu.sync_copy(idx_hbm.at[:, pl.ds(base_b, B_PER_W)], idx_v)

    lane = lax.iota(jnp.int32, 16)
    i_sub = lane >> 3
    i_row = lane & 7

    def unit_hj(u):
        return u // J_PER_W, u % J_PER_W

    def start_gather(u, b):
        h, jl = unit_hj(u)
        pltpu.async_copy(
            table_hbm.at[idx_v.at[h, pl.ds(jl * 128, 128)]],
            rows_v.at[b], gsems[b],
        )

    def wait_gather(b):
        pltpu.make_async_copy(
            table_hbm.at[idx_v.at[0, pl.ds(0, 128)]], rows_v.at[b], gsems[b]
        ).wait()

    def scatter_scale(b, pb):
        rbuf = rows_v.at[b]
        pbuf = patch_v.at[pb]

        @plsc.parallel_loop(0, 128, step=1, unroll=8)
        def _(r):
            rvec = jnp.full((16,), r, dtype=jnp.int32)
            for k in range(DIM // 16):
                v = rbuf[r, pl.ds(16 * k, 16)] * SCALE
                plsc.store_scatter(pbuf, [i_sub + 2 * k, i_row, rvec], v)

    def out_slice(u):
        h, jl = unit_hj(u)
        return out_hbm.at[h, :, jg_base + jl]

    for u0 in range(NBUF - 1):
        start_gather(u0, u0)

    def quad_body(t, carry):
        for b in range(NBUF):
            u = t * NBUF + b
            pb = b % 2

            @pl.when(u + NBUF - 1 < N_UNITS)
            def _():
                start_gather(u + NBUF - 1, (b + NBUF - 1) % NBUF)

            wait_gather(b)

            @pl.when(u >= 2)
            def _():
                pltpu.make_async_copy(
                    patch_v.at[pb, :, :, pl.ds(0, 128)], out_slice(u), osems[pb]
                ).wait()

            scatter_scale(b, pb)
            pltpu.async_copy(
                patch_v.at[pb, :, :, pl.ds(0, 128)], out_slice(u), osems[pb]
            )
        return carry

    lax.fori_loop(0, N_UNITS // NBUF, quad_body, 0)

    pltpu.make_async_copy(
        patch_v.at[0, :, :, pl.ds(0, 128)], out_slice(0), osems[0]
    ).wait()
    pltpu.make_async_copy(
        patch_v.at[1, :, :, pl.ds(0, 128)], out_slice(1), osems[1]
    ).wait()


def kernel(input, table):
    out5 = _embed_sc(input.T, table)
    o6 = out5.transpose(2, 4, 0, 1, 3)
    return o6.reshape(BATCH, HIST, DIM)


# speedup vs baseline: 3.2890x; 1.8865x over previous
import functools
import math

import jax
import jax.numpy as jnp
from jax import lax
from jax.experimental import pallas as pl
from jax.experimental.pallas import tpu as pltpu
from jax.experimental.pallas import tpu_sc as plsc

VOCAB = 1000000
DIM = 64
BATCH = 16384
HIST = 50
SCALE = math.sqrt(DIM)

_info = plsc.get_sparse_core_info()
NC = _info.num_cores
NS = _info.num_subcores
NW = NC * NS
B_PER_W = BATCH // NW
JCOLS = BATCH // 128
J_PER_W = JCOLS // NW
N_UNITS = HIST * J_PER_W
NBUF = 4

TCOLS = 7813
PAIR_ROWS = TCOLS * 64
PSTRIDE = 130
K1_ITERS = 246

_mesh = plsc.VectorSubcoreMesh(core_axis_name="c", subcore_axis_name="s")


@functools.partial(
    pl.kernel,
    mesh=_mesh,
    out_type=jax.ShapeDtypeStruct((PAIR_ROWS, 128), jnp.float32),
    scratch_types=[
        pltpu.VMEM((DIM, 128), jnp.float32),
        pltpu.VMEM((DIM, 128), jnp.float32),
        pltpu.VMEM((64 * PSTRIDE, ), jnp.float32),
        pltpu.VMEM((64 * PSTRIDE, ), jnp.float32),
        pltpu.VMEM((64, 128), jnp.float32),
        pltpu.VMEM((64, 128), jnp.float32),
        pltpu.SemaphoreType.DMA,
        pltpu.SemaphoreType.DMA,
        pltpu.SemaphoreType.DMA,
        pltpu.SemaphoreType.DMA,
    ],
    compiler_params=pltpu.CompilerParams(
        use_tc_tiling_on_sc=True, needs_layout_passes=False
    ),
)
def _table_sc(tabt_hbm, out_hbm, in0, in1, p0, p1, c0, c1, g0, g1, s0, s1):
    wid = lax.axis_index("s") * NC + lax.axis_index("c")
    inb = (in0, in1)
    pb = (p0, p1)
    cb = (c0, c1)
    gsems = (g0, g1)
    osems = (s0, s1)

    lane = lax.iota(jnp.int32, 16)
    parity_vec = (lane >> 1) * PSTRIDE + (lane & 1) * 64

    def jcol(i):
        return i * NW + wid

    def col0(i):
        return pl.multiple_of(jcol(i) * 128, 128)

    def start_in(i, b):
        pltpu.async_copy(
            tabt_hbm.at[:, pl.ds(col0(i), 128)], inb[b], gsems[b]
        )

    def wait_in(b):
        pltpu.make_async_copy(
            tabt_hbm.at[:, pl.ds(0, 128)], inb[b], gsems[b]
        ).wait()

    def transpose_unit(b):
        src = inb[b]
        pbuf = pb[b]
        cbuf = cb[b]

        @plsc.parallel_loop(0, DIM, step=1, unroll=4)
        def _(c):
            for g in range(8):
                v = src[c, pl.ds(16 * g, 16)]
                plsc.store_scatter(pbuf, [parity_vec + (1040 * g + c)], v)

        @plsc.parallel_loop(0, 64, step=1, unroll=4)
        def _(p):
            for m in range(8):
                cbuf[p, pl.ds(16 * m, 16)] = pbuf[pl.ds(p * PSTRIDE + 16 * m, 16)]

    def start_out(i, b):
        pltpu.async_copy(
            cb[b], out_hbm.at[pl.ds(pl.multiple_of(col0(i) // 2, 64), 64)],
            osems[b]
        )

    def wait_out(b):
        pltpu.make_async_copy(
            cb[b], out_hbm.at[pl.ds(0, 64)], osems[b]
        ).wait()

    start_in(0, 0)

    def pair_body(t, carry):
        for b in range(2):
            i = t * 2 + b

            @pl.when(jcol(i + 1) < TCOLS)
            def _():
                start_in(i + 1, 1 - b)

            @pl.when(jcol(i) < TCOLS)
            def _():
                wait_in(b)

                @pl.when(i >= 2)
                def _():
                    wait_out(b)

                transpose_unit(b)
                start_out(i, b)
        return carry

    lax.fori_loop(0, K1_ITERS // 2, pair_body, 0)
    wait_out(0)
    wait_out(1)


@functools.partial(
    pl.kernel,
    mesh=_mesh,
    out_type=jax.ShapeDtypeStruct((HIST, DIM // 8, JCOLS, 8, 128),
                                  jnp.float32),
    scratch_types=[
        pltpu.VMEM((HIST, B_PER_W), jnp.int32),
        pltpu.VMEM((NBUF, 128, DIM), jnp.float32),
        pltpu.VMEM((2, DIM // 8, 8, 129), jnp.float32),
        pltpu.SemaphoreType.DMA,
        pltpu.SemaphoreType.DMA,
        pltpu.SemaphoreType.DMA,
        pltpu.SemaphoreType.DMA,
        pltpu.SemaphoreType.DMA,
        pltpu.SemaphoreType.DMA,
    ],
    compiler_params=pltpu.CompilerParams(
        use_tc_tiling_on_sc=False, needs_layout_passes=False
    ),
)
def _embed_sc(idx_hbm, table_hbm, out_hbm, idx_v, rows_v, patch_v,
              g0, g1, g2, g3, s0, s1):
    wid = lax.axis_index("s") * NC + lax.axis_index("c")
    base_b = wid * B_PER_W
    jg_base = wid * J_PER_W
    gsems = (g0, g1, g2, g3)
    osems = (s0, s1)

    pltpu.sync_copy(idx_hbm.at[:, pl.ds(base_b, B_PER_W)], idx_v)

    lane = lax.iota(jnp.int32, 16)
    i_sub = lane >> 3
    i_row = lane & 7

    def unit_hj(u):
        return u // J_PER_W, u % J_PER_W

    def start_gather(u, b):
        h, jl = unit_hj(u)
        pltpu.async_copy(
            table_hbm.at[idx_v.at[h, pl.ds(jl * 128, 128)]],
            rows_v.at[b], gsems[b],
        )

    def wait_gather(b):
        pltpu.make_async_copy(
            table_hbm.at[idx_v.at[0, pl.ds(0, 128)]], rows_v.at[b], gsems[b]
        ).wait()

    def scatter_scale(b, pbi):
        rbuf = rows_v.at[b]
        pbuf = patch_v.at[pbi]

        @plsc.parallel_loop(0, 128, step=1, unroll=8)
        def _(r):
            rvec = jnp.full((16,), r, dtype=jnp.int32)
            for k in range(DIM // 16):
                v = rbuf[r, pl.ds(16 * k, 16)] * SCALE
                plsc.store_scatter(pbuf, [i_sub + 2 * k, i_row, rvec], v)

    def out_slice(u):
        h, jl = unit_hj(u)
        return out_hbm.at[h, :, jg_base + jl]

    for u0 in range(NBUF - 1):
        start_gather(u0, u0)

    def quad_body(t, carry):
        for b in range(NBUF):
            u = t * NBUF + b
            pbi = b % 2

            @pl.when(u + NBUF - 1 < N_UNITS)
            def _():
                start_gather(u + NBUF - 1, (b + NBUF - 1) % NBUF)

            wait_gather(b)

            @pl.when(u >= 2)
            def _():
                pltpu.make_async_copy(
                    patch_v.at[pbi, :, :, pl.ds(0, 128)], out_slice(u),
                    osems[pbi],
                ).wait()

            scatter_scale(b, pbi)
            pltpu.async_copy(
                patch_v.at[pbi, :, :, pl.ds(0, 128)], out_slice(u), osems[pbi]
            )
        return carry

    lax.fori_loop(0, N_UNITS // NBUF, quad_body, 0)

    pltpu.make_async_copy(
        patch_v.at[0, :, :, pl.ds(0, 128)], out_slice(0), osems[0]
    ).wait()
    pltpu.make_async_copy(
        patch_v.at[1, :, :, pl.ds(0, 128)], out_slice(1), osems[1]
    ).wait()


def kernel(input, table):
    t128 = _table_sc(table.T)
    tlin = t128.reshape(PAIR_ROWS * 2, DIM)
    out5 = _embed_sc(input.T, tlin)
    o6 = out5.transpose(2, 4, 0, 1, 3)
    return o6.reshape(BATCH, HIST, DIM)


# speedup vs baseline: 3.2993x; 1.0031x over previous
import functools
import math

import jax
import jax.numpy as jnp
from jax import lax
from jax.experimental import pallas as pl
from jax.experimental.pallas import tpu as pltpu
from jax.experimental.pallas import tpu_sc as plsc

VOCAB = 1000000
DIM = 64
BATCH = 16384
HIST = 50
SCALE = math.sqrt(DIM)

_info = plsc.get_sparse_core_info()
NC = _info.num_cores
NS = _info.num_subcores
NW = NC * NS
B_PER_W = BATCH // NW
JCOLS = BATCH // 128
J_PER_W = JCOLS // NW
N_UNITS = HIST * J_PER_W
NBUF = 4

TCOLS = 7813
PAIR_ROWS = TCOLS * 64
PSTRIDE = 129
K1_ITERS = 246

_mesh = plsc.VectorSubcoreMesh(core_axis_name="c", subcore_axis_name="s")


@functools.partial(
    pl.kernel,
    mesh=_mesh,
    out_type=jax.ShapeDtypeStruct((PAIR_ROWS, 128), jnp.float32),
    scratch_types=[
        pltpu.VMEM((DIM, 128), jnp.float32),
        pltpu.VMEM((DIM, 128), jnp.float32),
        pltpu.VMEM((128 * PSTRIDE, ), jnp.float32),
        pltpu.VMEM((128 * PSTRIDE, ), jnp.float32),
        pltpu.VMEM((64, 128), jnp.float32),
        pltpu.VMEM((64, 128), jnp.float32),
        pltpu.SemaphoreType.DMA,
        pltpu.SemaphoreType.DMA,
        pltpu.SemaphoreType.DMA,
        pltpu.SemaphoreType.DMA,
    ],
    compiler_params=pltpu.CompilerParams(
        use_tc_tiling_on_sc=True, needs_layout_passes=False
    ),
)
def _table_sc(tabt_hbm, out_hbm, in0, in1, p0, p1, c0, c1, g0, g1, s0, s1):
    wid = lax.axis_index("s") * NC + lax.axis_index("c")
    inb = (in0, in1)
    pb = (p0, p1)
    cb = (c0, c1)
    gsems = (g0, g1)
    osems = (s0, s1)

    lane = lax.iota(jnp.int32, 16)
    lane129 = lane * PSTRIDE

    def jcol(i):
        return i * NW + wid

    def col0(i):
        return pl.multiple_of(jcol(i) * 128, 128)

    def start_in(i, b):
        pltpu.async_copy(
            tabt_hbm.at[:, pl.ds(col0(i), 128)], inb[b], gsems[b]
        )

    def wait_in(b):
        pltpu.make_async_copy(
            tabt_hbm.at[:, pl.ds(0, 128)], inb[b], gsems[b]
        ).wait()

    def transpose_unit(b):
        src = inb[b]
        pbuf = pb[b]
        cbuf = cb[b]

        @plsc.parallel_loop(0, DIM, step=1, unroll=4)
        def _(c):
            for g in range(8):
                v = src[c, pl.ds(16 * g, 16)]
                plsc.store_scatter(
                    pbuf, [lane129 + (16 * PSTRIDE * g + c)], v
                )

        @plsc.parallel_loop(0, 128, step=1, unroll=4)
        def _(j):
            p = j >> 1
            zoff = (j & 1) * 64
            for m in range(4):
                cbuf[p, pl.ds(zoff + 16 * m, 16)] = (
                    pbuf[pl.ds(j * PSTRIDE + 16 * m, 16)]
                )

    def start_out(i, b):
        pltpu.async_copy(
            cb[b], out_hbm.at[pl.ds(pl.multiple_of(col0(i) // 2, 64), 64)],
            osems[b]
        )

    def wait_out(b):
        pltpu.make_async_copy(
            cb[b], out_hbm.at[pl.ds(0, 64)], osems[b]
        ).wait()

    start_in(0, 0)

    def pair_body(t, carry):
        for b in range(2):
            i = t * 2 + b

            @pl.when(jcol(i + 1) < TCOLS)
            def _():
                start_in(i + 1, 1 - b)

            @pl.when(jcol(i) < TCOLS)
            def _():
                wait_in(b)

                @pl.when(i >= 2)
                def _():
                    wait_out(b)

                transpose_unit(b)
                start_out(i, b)
        return carry

    lax.fori_loop(0, K1_ITERS // 2, pair_body, 0)
    wait_out(0)
    wait_out(1)


@functools.partial(
    pl.kernel,
    mesh=_mesh,
    out_type=jax.ShapeDtypeStruct((HIST, DIM // 8, JCOLS, 8, 128),
                                  jnp.float32),
    scratch_types=[
        pltpu.VMEM((HIST, B_PER_W), jnp.int32),
        pltpu.VMEM((NBUF, 128, DIM), jnp.float32),
        pltpu.VMEM((2, DIM // 8, 8, 129), jnp.float32),
        pltpu.SemaphoreType.DMA,
        pltpu.SemaphoreType.DMA,
        pltpu.SemaphoreType.DMA,
        pltpu.SemaphoreType.DMA,
        pltpu.SemaphoreType.DMA,
        pltpu.SemaphoreType.DMA,
    ],
    compiler_params=pltpu.CompilerParams(
        use_tc_tiling_on_sc=False, needs_layout_passes=False
    ),
)
def _embed_sc(idx_hbm, table_hbm, out_hbm, idx_v, rows_v, patch_v,
              g0, g1, g2, g3, s0, s1):
    wid = lax.axis_index("s") * NC + lax.axis_index("c")
    base_b = wid * B_PER_W
    jg_base = wid * J_PER_W
    gsems = (g0, g1, g2, g3)
    osems = (s0, s1)

    pltpu.sync_copy(idx_hbm.at[:, pl.ds(base_b, B_PER_W)], idx_v)

    lane = lax.iota(jnp.int32, 16)
    i_sub = lane >> 3
    i_row = lane & 7

    def unit_hj(u):
        return u // J_PER_W, u % J_PER_W

    def start_gather(u, b):
        h, jl = unit_hj(u)
        pltpu.async_copy(
            table_hbm.at[idx_v.at[h, pl.ds(jl * 128, 128)]],
            rows_v.at[b], gsems[b],
        )

    def wait_gather(b):
        pltpu.make_async_copy(
            table_hbm.at[idx_v.at[0, pl.ds(0, 128)]], rows_v.at[b], gsems[b]
        ).wait()

    def scatter_scale(b, pbi):
        rbuf = rows_v.at[b]
        pbuf = patch_v.at[pbi]

        @plsc.parallel_loop(0, 128, step=1, unroll=8)
        def _(r):
            rvec = jnp.full((16,), r, dtype=jnp.int32)
            for k in range(DIM // 16):
                v = rbuf[r, pl.ds(16 * k, 16)] * SCALE
                plsc.store_scatter(pbuf, [i_sub + 2 * k, i_row, rvec], v)

    def out_slice(u):
        h, jl = unit_hj(u)
        return out_hbm.at[h, :, jg_base + jl]

    for u0 in range(NBUF - 1):
        start_gather(u0, u0)

    def quad_body(t, carry):
        for b in range(NBUF):
            u = t * NBUF + b
            pbi = b % 2

            @pl.when(u + NBUF - 1 < N_UNITS)
            def _():
                start_gather(u + NBUF - 1, (b + NBUF - 1) % NBUF)

            wait_gather(b)

            @pl.when(u >= 2)
            def _():
                pltpu.make_async_copy(
                    patch_v.at[pbi, :, :, pl.ds(0, 128)], out_slice(u),
                    osems[pbi],
                ).wait()

            scatter_scale(b, pbi)
            pltpu.async_copy(
                patch_v.at[pbi, :, :, pl.ds(0, 128)], out_slice(u), osems[pbi]
            )
        return carry

    lax.fori_loop(0, N_UNITS // NBUF, quad_body, 0)

    pltpu.make_async_copy(
        patch_v.at[0, :, :, pl.ds(0, 128)], out_slice(0), osems[0]
    ).wait()
    pltpu.make_async_copy(
        patch_v.at[1, :, :, pl.ds(0, 128)], out_slice(1), osems[1]
    ).wait()


def kernel(input, table):
    t128 = _table_sc(table.T)
    tlin = t128.reshape(PAIR_ROWS * 2, DIM)
    out5 = _embed_sc(input.T, tlin)
    o6 = out5.transpose(2, 4, 0, 1, 3)
    return o6.reshape(BATCH, HIST, DIM)
